# Initial kernel scaffold; baseline (speedup 1.0000x reference)
import jax
import jax.numpy as jnp
from jax.experimental import pallas as pl


def kernel(x, edge_index, W_enc, b_enc, W_layers, b_layers, W_dec, b_dec):
    raise NotImplementedError("write your pallas kernel here")


# speedup vs baseline: 16.3748x; 16.3748x over previous
import functools

import jax
import jax.numpy as jnp
from jax import lax
from jax.experimental import pallas as pl
from jax.experimental.pallas import tpu as pltpu
from jax.experimental.pallas import tpu_sc as plsc

N = 10000
H = 128
E = 320000

NC = 2
NS = 16
NW = NC * NS
EW = E // NW
C = 80
NCHUNK = EW // C
RPT = 624
REM = N - NS * RPT

R = 1000
G = N // R


def _sc_aggregate(t, ei0, ei1):
    mesh = plsc.VectorSubcoreMesh(core_axis_name="c", subcore_axis_name="s")

    @functools.partial(
        pl.kernel,
        out_type=jax.ShapeDtypeStruct((2 * N, H), jnp.float32),
        mesh=mesh,
        scratch_types=[
            pltpu.VMEM((EW,), jnp.int32),
            pltpu.VMEM((EW,), jnp.int32),
            pltpu.VMEM((C, H), jnp.float32),
            pltpu.VMEM((C, H), jnp.float32),
            pltpu.VMEM((C,), jnp.int32),
            pltpu.VMEM((C,), jnp.int32),
            pltpu.VMEM_SHARED((N, H), jnp.float32),
            pltpu.SemaphoreType.DMA,
            pltpu.SemaphoreType.DMA,
        ],
    )
    def agg(t_hbm, ei0_hbm, ei1_hbm, out_hbm, eiA, eiB, rows0, rows1,
            dst0, dst1, acc, sem0, sem1):
        c = lax.axis_index("c")
        s = lax.axis_index("s")
        wid = s * NC + c

        zeros16 = jnp.zeros((16,), jnp.float32)

        def zrow(r, carry):
            for j in range(H // 16):
                rows0[r, pl.ds(j * 16, 16)] = zeros16
            return carry

        lax.fori_loop(0, C, zrow, 0)
        for k in range(RPT // C):
            pltpu.sync_copy(rows0, acc.at[pl.ds(s * RPT + k * C, C)])
        pltpu.sync_copy(rows0.at[pl.ds(0, RPT % C)],
                        acc.at[pl.ds(s * RPT + (RPT // C) * C, RPT % C)])

        @pl.when(s == NS - 1)
        def _zero_rem():
            pltpu.sync_copy(rows0.at[pl.ds(0, REM)],
                            acc.at[pl.ds(NS * RPT, REM)])

        plsc.subcore_barrier()

        pltpu.sync_copy(ei0_hbm.at[pl.ds(wid * EW, EW)], eiA)
        pltpu.sync_copy(ei1_hbm.at[pl.ds(wid * EW, EW)], eiB)

        def gstart(src_all, g, rows, sem):
            pltpu.async_copy(t_hbm.at[src_all.at[pl.ds(g * C, C)]], rows, sem)

        def gwait(src_all, g, rows, sem):
            pltpu.make_async_copy(
                t_hbm.at[src_all.at[pl.ds(g * C, C)]], rows, sem).wait()

        def process(src_all, dst_all, g, rows, sem, dbuf):
            for v in range(C // 16):
                dbuf[pl.ds(v * 16, 16)] = dst_all[pl.ds(g * C + v * 16, 16)]
            gwait(src_all, g, rows, sem)
            pltpu.sync_copy(rows, acc.at[dbuf], add=True)

        for (src_all, dst_all) in ((eiA, eiB), (eiB, eiA)):
            gstart(src_all, 0, rows0, sem0)

            def body(i, carry):
                g = 2 * i
                gstart(src_all, g + 1, rows1, sem1)
                process(src_all, dst_all, g, rows0, sem0, dst0)
                gstart(src_all, g + 2, rows0, sem0)
                process(src_all, dst_all, g + 1, rows1, sem1, dst1)
                return carry

            lax.fori_loop(0, (NCHUNK - 1) // 2, body, 0)
            process(src_all, dst_all, NCHUNK - 1, rows0, sem0, dst0)

        plsc.subcore_barrier()
        pltpu.sync_copy(acc.at[pl.ds(s * RPT, RPT)],
                        out_hbm.at[pl.ds(c * N + s * RPT, RPT)])

        @pl.when(s == NS - 1)
        def _out_rem():
            pltpu.sync_copy(acc.at[pl.ds(NS * RPT, REM)],
                            out_hbm.at[pl.ds(c * N + NS * RPT, REM)])

    return agg(t, ei0, ei1)


def _tc_encode(x, We, be, W0, b0):
    D = x.shape[1]

    def body(x_ref, we_ref, be_ref, w0_ref, b0_ref, x0_ref, t_ref):
        x0 = jnp.dot(x_ref[...], we_ref[...],
                     preferred_element_type=jnp.float32) + be_ref[...]
        x0_ref[...] = x0
        t_ref[...] = jnp.dot(x0, w0_ref[...],
                             preferred_element_type=jnp.float32) + b0_ref[...]

    return pl.pallas_call(
        body,
        grid=(G,),
        in_specs=[
            pl.BlockSpec((R, D), lambda i: (i, 0)),
            pl.BlockSpec((D, H), lambda i: (0, 0)),
            pl.BlockSpec((1, H), lambda i: (0, 0)),
            pl.BlockSpec((H, H), lambda i: (0, 0)),
            pl.BlockSpec((1, H), lambda i: (0, 0)),
        ],
        out_specs=(
            pl.BlockSpec((R, H), lambda i: (i, 0)),
            pl.BlockSpec((R, H), lambda i: (i, 0)),
        ),
        out_shape=(
            jax.ShapeDtypeStruct((N, H), jnp.float32),
            jax.ShapeDtypeStruct((N, H), jnp.float32),
        ),
    )(x, We, be.reshape(1, H), W0, b0.reshape(1, H))


def _tc_mid(P, x0, W, b):
    def body(p0_ref, p1_ref, x0_ref, w_ref, b_ref, t_ref):
        h = jnp.maximum(p0_ref[...] + p1_ref[...] + x0_ref[...], 0.0)
        t_ref[...] = jnp.dot(h, w_ref[...],
                             preferred_element_type=jnp.float32) + b_ref[...]

    return pl.pallas_call(
        body,
        grid=(G,),
        in_specs=[
            pl.BlockSpec((R, H), lambda i: (i, 0)),
            pl.BlockSpec((R, H), lambda i: (i + G, 0)),
            pl.BlockSpec((R, H), lambda i: (i, 0)),
            pl.BlockSpec((H, H), lambda i: (0, 0)),
            pl.BlockSpec((1, H), lambda i: (0, 0)),
        ],
        out_specs=pl.BlockSpec((R, H), lambda i: (i, 0)),
        out_shape=jax.ShapeDtypeStruct((N, H), jnp.float32),
    )(P, P, x0, W, b.reshape(1, H))


def _tc_final(P, x0, Wd, bd):
    def body(p0_ref, p1_ref, x0_ref, wd_ref, bd_ref, o_ref):
        h = jnp.maximum(p0_ref[...] + p1_ref[...] + x0_ref[...], 0.0)
        o_ref[...] = jnp.sum(h * wd_ref[...], axis=1,
                             keepdims=True) + bd_ref[...]

    return pl.pallas_call(
        body,
        grid=(G,),
        in_specs=[
            pl.BlockSpec((R, H), lambda i: (i, 0)),
            pl.BlockSpec((R, H), lambda i: (i + G, 0)),
            pl.BlockSpec((R, H), lambda i: (i, 0)),
            pl.BlockSpec((1, H), lambda i: (0, 0)),
            pl.BlockSpec((1, 1), lambda i: (0, 0)),
        ],
        out_specs=pl.BlockSpec((R, 1), lambda i: (i, 0)),
        out_shape=jax.ShapeDtypeStruct((N, 1), jnp.float32),
    )(P, P, x0, Wd.reshape(1, H), bd.reshape(1, 1))


def kernel(x, edge_index, W_enc, b_enc, W_layers, b_layers, W_dec, b_dec):
    L = W_layers.shape[0]
    ei0 = edge_index[0]
    ei1 = edge_index[1]
    x0, t = _tc_encode(x, W_enc, b_enc, W_layers[0], b_layers[0])
    out = None
    for i in range(L):
        P = _sc_aggregate(t, ei0, ei1)
        if i + 1 < L:
            t = _tc_mid(P, x0, W_layers[i + 1], b_layers[i + 1])
        else:
            out = _tc_final(P, x0, W_dec, b_dec)
    return out


# speedup vs baseline: 19.1825x; 1.1715x over previous
import functools

import jax
import jax.numpy as jnp
from jax import lax
from jax.experimental import pallas as pl
from jax.experimental.pallas import tpu as pltpu
from jax.experimental.pallas import tpu_sc as plsc

N = 10000
H = 128
E = 320000

NC = 2
NS = 16
NW = NC * NS
EW = E // NW
C = 80
NCHUNK = EW // C
RPT = 624
REM = N - NS * RPT

R = 1000
G = N // R


def _sc_aggregate(t, ei0, ei1):
    mesh = plsc.VectorSubcoreMesh(core_axis_name="c", subcore_axis_name="s")

    @functools.partial(
        pl.kernel,
        out_type=jax.ShapeDtypeStruct((2 * N, H), jnp.float32),
        mesh=mesh,
        scratch_types=[
            pltpu.VMEM((EW,), jnp.int32),
            pltpu.VMEM((C, H), jnp.float32),
            pltpu.VMEM((C, H), jnp.float32),
            pltpu.VMEM((C, H), jnp.float32),
            pltpu.VMEM((C,), jnp.int32),
            pltpu.VMEM((C,), jnp.int32),
            pltpu.VMEM((C,), jnp.int32),
            pltpu.VMEM_SHARED((N, H), jnp.float32),
            pltpu.SemaphoreType.DMA,
            pltpu.SemaphoreType.DMA,
            pltpu.SemaphoreType.DMA,
            pltpu.SemaphoreType.DMA,
            pltpu.SemaphoreType.DMA,
            pltpu.SemaphoreType.DMA,
            pltpu.SemaphoreType.DMA,
            pltpu.SemaphoreType.DMA,
            pltpu.SemaphoreType.DMA,
        ],
    )
    def agg(t_hbm, ei0_hbm, ei1_hbm, out_hbm, eiA, rows0, rows1, rows2,
            dst0, dst1, dst2, acc,
            gsem0, gsem1, gsem2, ssem0, ssem1, ssem2, dsem0, dsem1, dsem2):
        c = lax.axis_index("c")
        s = lax.axis_index("s")
        wid = s * NC + c

        zeros16 = jnp.zeros((16,), jnp.float32)

        def zrow(r, carry):
            for j in range(H // 16):
                rows0[r, pl.ds(j * 16, 16)] = zeros16
            return carry

        lax.fori_loop(0, C, zrow, 0)
        for k in range(RPT // C):
            pltpu.sync_copy(rows0, acc.at[pl.ds(s * RPT + k * C, C)])
        pltpu.sync_copy(rows0.at[pl.ds(0, RPT % C)],
                        acc.at[pl.ds(s * RPT + (RPT // C) * C, RPT % C)])

        @pl.when(s == NS - 1)
        def _zero_rem():
            pltpu.sync_copy(rows0.at[pl.ds(0, REM)],
                            acc.at[pl.ds(NS * RPT, REM)])

        plsc.subcore_barrier()

        rows = (rows0, rows1, rows2)
        dst = (dst0, dst1, dst2)
        gsem = (gsem0, gsem1, gsem2)
        ssem = (ssem0, ssem1, ssem2)
        dsem = (dsem0, dsem1, dsem2)

        def gstart(g, b):
            pltpu.async_copy(
                t_hbm.at[eiA.at[pl.ds(g * C, C)]], rows[b], gsem[b])

        def gwait(g, b):
            pltpu.make_async_copy(
                t_hbm.at[eiA.at[pl.ds(g * C, C)]], rows[b], gsem[b]).wait()

        def sstart(b):
            pltpu.async_copy(rows[b], acc.at[dst[b]], ssem[b], add=True)

        def swait(b):
            pltpu.make_async_copy(rows[b], acc.at[dst[b]], ssem[b]).wait()

        for (src_hbm, dst_hbm) in ((ei0_hbm, ei1_hbm), (ei1_hbm, ei0_hbm)):
            pltpu.sync_copy(src_hbm.at[pl.ds(wid * EW, EW)], eiA)

            def dstart(g, b, dst_hbm=dst_hbm):
                pltpu.async_copy(
                    dst_hbm.at[pl.ds(wid * EW + g * C, C)], dst[b], dsem[b])

            def dwait(g, b, dst_hbm=dst_hbm):
                pltpu.make_async_copy(
                    dst_hbm.at[pl.ds(wid * EW + g * C, C)], dst[b],
                    dsem[b]).wait()

            def process(g, b, wait_prev=True):
                dwait(g, b)
                gwait(g, b)
                if wait_prev:
                    swait((b + 2) % 3)
                sstart(b)

            def launch(g, b):
                dstart(g, b)
                gstart(g, b)

            launch(0, 0)
            launch(1, 1)
            process(0, 0, wait_prev=False)
            launch(2, 2)
            process(1, 1)
            launch(3, 0)
            process(2, 2)
            launch(4, 1)

            def body(i, carry):
                g = 3 * i
                for b in range(3):
                    process(g + b, b)
                    launch(g + b + 2, (b + 2) % 3)
                return carry

            lax.fori_loop(1, (NCHUNK - 2) // 3, body, 0)
            process(NCHUNK - 2, 0)
            process(NCHUNK - 1, 1)
            swait(1)

        plsc.subcore_barrier()
        pltpu.sync_copy(acc.at[pl.ds(s * RPT, RPT)],
                        out_hbm.at[pl.ds(c * N + s * RPT, RPT)])

        @pl.when(s == NS - 1)
        def _out_rem():
            pltpu.sync_copy(acc.at[pl.ds(NS * RPT, REM)],
                            out_hbm.at[pl.ds(c * N + NS * RPT, REM)])

    return agg(t, ei0, ei1)


def _tc_encode(x, We, be, W0, b0):
    D = x.shape[1]

    def body(x_ref, we_ref, be_ref, w0_ref, b0_ref, x0_ref, t_ref):
        x0 = jnp.dot(x_ref[...], we_ref[...],
                     preferred_element_type=jnp.float32) + be_ref[...]
        x0_ref[...] = x0
        t_ref[...] = jnp.dot(x0, w0_ref[...],
                             preferred_element_type=jnp.float32) + b0_ref[...]

    return pl.pallas_call(
        body,
        grid=(G,),
        in_specs=[
            pl.BlockSpec((R, D), lambda i: (i, 0)),
            pl.BlockSpec((D, H), lambda i: (0, 0)),
            pl.BlockSpec((1, H), lambda i: (0, 0)),
            pl.BlockSpec((H, H), lambda i: (0, 0)),
            pl.BlockSpec((1, H), lambda i: (0, 0)),
        ],
        out_specs=(
            pl.BlockSpec((R, H), lambda i: (i, 0)),
            pl.BlockSpec((R, H), lambda i: (i, 0)),
        ),
        out_shape=(
            jax.ShapeDtypeStruct((N, H), jnp.float32),
            jax.ShapeDtypeStruct((N, H), jnp.float32),
        ),
    )(x, We, be.reshape(1, H), W0, b0.reshape(1, H))


def _tc_mid(P, x0, W, b):
    def body(p0_ref, p1_ref, x0_ref, w_ref, b_ref, t_ref):
        h = jnp.maximum(p0_ref[...] + p1_ref[...] + x0_ref[...], 0.0)
        t_ref[...] = jnp.dot(h, w_ref[...],
                             preferred_element_type=jnp.float32) + b_ref[...]

    return pl.pallas_call(
        body,
        grid=(G,),
        in_specs=[
            pl.BlockSpec((R, H), lambda i: (i, 0)),
            pl.BlockSpec((R, H), lambda i: (i + G, 0)),
            pl.BlockSpec((R, H), lambda i: (i, 0)),
            pl.BlockSpec((H, H), lambda i: (0, 0)),
            pl.BlockSpec((1, H), lambda i: (0, 0)),
        ],
        out_specs=pl.BlockSpec((R, H), lambda i: (i, 0)),
        out_shape=jax.ShapeDtypeStruct((N, H), jnp.float32),
    )(P, P, x0, W, b.reshape(1, H))


def _tc_final(P, x0, Wd, bd):
    def body(p0_ref, p1_ref, x0_ref, wd_ref, bd_ref, o_ref):
        h = jnp.maximum(p0_ref[...] + p1_ref[...] + x0_ref[...], 0.0)
        o_ref[...] = jnp.sum(h * wd_ref[...], axis=1,
                             keepdims=True) + bd_ref[...]

    return pl.pallas_call(
        body,
        grid=(G,),
        in_specs=[
            pl.BlockSpec((R, H), lambda i: (i, 0)),
            pl.BlockSpec((R, H), lambda i: (i + G, 0)),
            pl.BlockSpec((R, H), lambda i: (i, 0)),
            pl.BlockSpec((1, H), lambda i: (0, 0)),
            pl.BlockSpec((1, 1), lambda i: (0, 0)),
        ],
        out_specs=pl.BlockSpec((R, 1), lambda i: (i, 0)),
        out_shape=jax.ShapeDtypeStruct((N, 1), jnp.float32),
    )(P, P, x0, Wd.reshape(1, H), bd.reshape(1, 1))


def kernel(x, edge_index, W_enc, b_enc, W_layers, b_layers, W_dec, b_dec):
    L = W_layers.shape[0]
    ei0 = edge_index[0]
    ei1 = edge_index[1]
    x0, t = _tc_encode(x, W_enc, b_enc, W_layers[0], b_layers[0])
    out = None
    for i in range(L):
        P = _sc_aggregate(t, ei0, ei1)
        if i + 1 < L:
            t = _tc_mid(P, x0, W_layers[i + 1], b_layers[i + 1])
        else:
            out = _tc_final(P, x0, W_dec, b_dec)
    return out


# speedup vs baseline: 19.2021x; 1.0010x over previous
import functools

import jax
import jax.numpy as jnp
from jax import lax
from jax.experimental import pallas as pl
from jax.experimental.pallas import tpu as pltpu
from jax.experimental.pallas import tpu_sc as plsc

N = 10000
H = 128
E = 320000

NC = 2
NS = 16
NW = NC * NS
EW = E // NW
C = 80
NCHUNK = EW // C
RPT = 624
REM = N - NS * RPT

R = 1000
G = N // R


def _sc_aggregate(t, ei0, ei1):
    mesh = plsc.VectorSubcoreMesh(core_axis_name="c", subcore_axis_name="s")

    @functools.partial(
        pl.kernel,
        out_type=jax.ShapeDtypeStruct((2 * N, H), jnp.float32),
        mesh=mesh,
        scratch_types=[
            pltpu.VMEM((EW,), jnp.int32),
            pltpu.VMEM((C, H), jnp.float32),
            pltpu.VMEM((C, H), jnp.float32),
            pltpu.VMEM((C, H), jnp.float32),
            pltpu.VMEM((C,), jnp.int32),
            pltpu.VMEM((C,), jnp.int32),
            pltpu.VMEM((C,), jnp.int32),
            pltpu.VMEM_SHARED((N, H), jnp.float32),
            pltpu.SemaphoreType.DMA,
            pltpu.SemaphoreType.DMA,
            pltpu.SemaphoreType.DMA,
            pltpu.SemaphoreType.DMA,
            pltpu.SemaphoreType.DMA,
            pltpu.SemaphoreType.DMA,
            pltpu.SemaphoreType.DMA,
            pltpu.SemaphoreType.DMA,
            pltpu.SemaphoreType.DMA,
            pltpu.SemaphoreType.DMA,
            pltpu.SemaphoreType.DMA,
            pltpu.SemaphoreType.DMA,
        ],
    )
    def agg(t_hbm, ei0_hbm, ei1_hbm, out_hbm, eiA, rows0, rows1, rows2,
            dst0, dst1, dst2, acc,
            gsem0, gsem1, gsem2, ssem0, ssem1, ssem2, dsem0, dsem1, dsem2,
            hsem0, hsem1, hsem2):
        c = lax.axis_index("c")
        s = lax.axis_index("s")
        wid = s * NC + c

        zeros16 = jnp.zeros((16,), jnp.float32)

        def zrow(r, carry):
            for j in range(H // 16):
                rows0[r, pl.ds(j * 16, 16)] = zeros16
            return carry

        lax.fori_loop(0, C, zrow, 0)
        for k in range(RPT // C):
            pltpu.sync_copy(rows0, acc.at[pl.ds(s * RPT + k * C, C)])
        pltpu.sync_copy(rows0.at[pl.ds(0, RPT % C)],
                        acc.at[pl.ds(s * RPT + (RPT // C) * C, RPT % C)])

        @pl.when(s == NS - 1)
        def _zero_rem():
            pltpu.sync_copy(rows0.at[pl.ds(0, REM)],
                            acc.at[pl.ds(NS * RPT, REM)])

        plsc.subcore_barrier()

        rows = (rows0, rows1, rows2)
        dst = (dst0, dst1, dst2)
        gsem = (gsem0, gsem1, gsem2)
        ssem = (ssem0, ssem1, ssem2)
        dsem = (dsem0, dsem1, dsem2)
        hsem = (hsem0, hsem1, hsem2)
        C2 = C // 2

        def gstart(g, b):
            pltpu.async_copy(
                t_hbm.at[eiA.at[pl.ds(g * C, C2)]],
                rows[b].at[pl.ds(0, C2)], gsem[b])
            pltpu.async_copy(
                t_hbm.at[eiA.at[pl.ds(g * C + C2, C2)]],
                rows[b].at[pl.ds(C2, C2)], hsem[b])

        def gwait(g, b):
            pltpu.make_async_copy(
                t_hbm.at[eiA.at[pl.ds(g * C, C2)]],
                rows[b].at[pl.ds(0, C2)], gsem[b]).wait()
            pltpu.make_async_copy(
                t_hbm.at[eiA.at[pl.ds(g * C + C2, C2)]],
                rows[b].at[pl.ds(C2, C2)], hsem[b]).wait()

        def sstart(b):
            pltpu.async_copy(rows[b], acc.at[dst[b]], ssem[b], add=True)

        def swait(b):
            pltpu.make_async_copy(rows[b], acc.at[dst[b]], ssem[b]).wait()

        for (src_hbm, dst_hbm) in ((ei0_hbm, ei1_hbm), (ei1_hbm, ei0_hbm)):
            pltpu.sync_copy(src_hbm.at[pl.ds(wid * EW, EW)], eiA)

            def dstart(g, b, dst_hbm=dst_hbm):
                pltpu.async_copy(
                    dst_hbm.at[pl.ds(wid * EW + g * C, C)], dst[b], dsem[b])

            def dwait(g, b, dst_hbm=dst_hbm):
                pltpu.make_async_copy(
                    dst_hbm.at[pl.ds(wid * EW + g * C, C)], dst[b],
                    dsem[b]).wait()

            def process(g, b, wait_prev=True):
                dwait(g, b)
                gwait(g, b)
                if wait_prev:
                    swait((b + 2) % 3)
                sstart(b)

            def launch(g, b):
                dstart(g, b)
                gstart(g, b)

            launch(0, 0)
            launch(1, 1)
            process(0, 0, wait_prev=False)
            launch(2, 2)
            process(1, 1)
            launch(3, 0)
            process(2, 2)
            launch(4, 1)

            def body(i, carry):
                g = 3 * i
                for b in range(3):
                    process(g + b, b)
                    launch(g + b + 2, (b + 2) % 3)
                return carry

            lax.fori_loop(1, (NCHUNK - 2) // 3, body, 0)
            process(NCHUNK - 2, 0)
            process(NCHUNK - 1, 1)
            swait(1)

        plsc.subcore_barrier()
        pltpu.sync_copy(acc.at[pl.ds(s * RPT, RPT)],
                        out_hbm.at[pl.ds(c * N + s * RPT, RPT)])

        @pl.when(s == NS - 1)
        def _out_rem():
            pltpu.sync_copy(acc.at[pl.ds(NS * RPT, REM)],
                            out_hbm.at[pl.ds(c * N + NS * RPT, REM)])

    return agg(t, ei0, ei1)


def _tc_encode(x, We, be, W0, b0):
    D = x.shape[1]

    def body(x_ref, we_ref, be_ref, w0_ref, b0_ref, x0_ref, t_ref):
        x0 = jnp.dot(x_ref[...], we_ref[...],
                     preferred_element_type=jnp.float32) + be_ref[...]
        x0_ref[...] = x0
        t_ref[...] = jnp.dot(x0, w0_ref[...],
                             preferred_element_type=jnp.float32) + b0_ref[...]

    return pl.pallas_call(
        body,
        grid=(G,),
        in_specs=[
            pl.BlockSpec((R, D), lambda i: (i, 0)),
            pl.BlockSpec((D, H), lambda i: (0, 0)),
            pl.BlockSpec((1, H), lambda i: (0, 0)),
            pl.BlockSpec((H, H), lambda i: (0, 0)),
            pl.BlockSpec((1, H), lambda i: (0, 0)),
        ],
        out_specs=(
            pl.BlockSpec((R, H), lambda i: (i, 0)),
            pl.BlockSpec((R, H), lambda i: (i, 0)),
        ),
        out_shape=(
            jax.ShapeDtypeStruct((N, H), jnp.float32),
            jax.ShapeDtypeStruct((N, H), jnp.float32),
        ),
    )(x, We, be.reshape(1, H), W0, b0.reshape(1, H))


def _tc_mid(P, x0, W, b):
    def body(p0_ref, p1_ref, x0_ref, w_ref, b_ref, t_ref):
        h = jnp.maximum(p0_ref[...] + p1_ref[...] + x0_ref[...], 0.0)
        t_ref[...] = jnp.dot(h, w_ref[...],
                             preferred_element_type=jnp.float32) + b_ref[...]

    return pl.pallas_call(
        body,
        grid=(G,),
        in_specs=[
            pl.BlockSpec((R, H), lambda i: (i, 0)),
            pl.BlockSpec((R, H), lambda i: (i + G, 0)),
            pl.BlockSpec((R, H), lambda i: (i, 0)),
            pl.BlockSpec((H, H), lambda i: (0, 0)),
            pl.BlockSpec((1, H), lambda i: (0, 0)),
        ],
        out_specs=pl.BlockSpec((R, H), lambda i: (i, 0)),
        out_shape=jax.ShapeDtypeStruct((N, H), jnp.float32),
    )(P, P, x0, W, b.reshape(1, H))


def _tc_final(P, x0, Wd, bd):
    def body(p0_ref, p1_ref, x0_ref, wd_ref, bd_ref, o_ref):
        h = jnp.maximum(p0_ref[...] + p1_ref[...] + x0_ref[...], 0.0)
        o_ref[...] = jnp.sum(h * wd_ref[...], axis=1,
                             keepdims=True) + bd_ref[...]

    return pl.pallas_call(
        body,
        grid=(G,),
        in_specs=[
            pl.BlockSpec((R, H), lambda i: (i, 0)),
            pl.BlockSpec((R, H), lambda i: (i + G, 0)),
            pl.BlockSpec((R, H), lambda i: (i, 0)),
            pl.BlockSpec((1, H), lambda i: (0, 0)),
            pl.BlockSpec((1, 1), lambda i: (0, 0)),
        ],
        out_specs=pl.BlockSpec((R, 1), lambda i: (i, 0)),
        out_shape=jax.ShapeDtypeStruct((N, 1), jnp.float32),
    )(P, P, x0, Wd.reshape(1, H), bd.reshape(1, 1))


def kernel(x, edge_index, W_enc, b_enc, W_layers, b_layers, W_dec, b_dec):
    L = W_layers.shape[0]
    ei0 = edge_index[0]
    ei1 = edge_index[1]
    x0, t = _tc_encode(x, W_enc, b_enc, W_layers[0], b_layers[0])
    out = None
    for i in range(L):
        P = _sc_aggregate(t, ei0, ei1)
        if i + 1 < L:
            t = _tc_mid(P, x0, W_layers[i + 1], b_layers[i + 1])
        else:
            out = _tc_final(P, x0, W_dec, b_dec)
    return out


# speedup vs baseline: 20.5454x; 1.0700x over previous
import functools

import jax
import jax.numpy as jnp
from jax import lax
from jax.experimental import pallas as pl
from jax.experimental.pallas import tpu as pltpu
from jax.experimental.pallas import tpu_sc as plsc

N = 10000
H = 128
E = 320000

NC = 2
NS = 16
NW = NC * NS
EW = E // NW
C = 40
NCHUNK = EW // C
RB = 6
RPT = 624
REM = N - NS * RPT

R = 1000
G = N // R


def _sc_aggregate(t, ei0, ei1):
    mesh = plsc.VectorSubcoreMesh(core_axis_name="c", subcore_axis_name="s")

    @functools.partial(
        pl.kernel,
        out_type=jax.ShapeDtypeStruct((2 * N, H), jnp.float32),
        mesh=mesh,
        scratch_types=(
            [pltpu.VMEM((EW,), jnp.int32)]
            + [pltpu.VMEM((C, H), jnp.float32)] * RB
            + [pltpu.VMEM((C,), jnp.int32)] * RB
            + [pltpu.VMEM_SHARED((N, H), jnp.float32)]
            + [pltpu.SemaphoreType.DMA] * (3 * RB)
        ),
    )
    def agg(t_hbm, ei0_hbm, ei1_hbm, out_hbm, *scr):
        eiA = scr[0]
        rows = scr[1:1 + RB]
        dst = scr[1 + RB:1 + 2 * RB]
        acc = scr[1 + 2 * RB]
        gsem = scr[2 + 2 * RB:2 + 3 * RB]
        ssem = scr[2 + 3 * RB:2 + 4 * RB]
        dsem = scr[2 + 4 * RB:2 + 5 * RB]

        c = lax.axis_index("c")
        s = lax.axis_index("s")
        wid = s * NC + c

        zeros16 = jnp.zeros((16,), jnp.float32)

        def zrow(r, carry):
            for j in range(H // 16):
                rows[0][r, pl.ds(j * 16, 16)] = zeros16
            return carry

        lax.fori_loop(0, C, zrow, 0)
        for k in range(RPT // C):
            pltpu.sync_copy(rows[0], acc.at[pl.ds(s * RPT + k * C, C)])
        pltpu.sync_copy(rows[0].at[pl.ds(0, RPT % C)],
                        acc.at[pl.ds(s * RPT + (RPT // C) * C, RPT % C)])

        @pl.when(s == NS - 1)
        def _zero_rem():
            pltpu.sync_copy(rows[0].at[pl.ds(0, REM)],
                            acc.at[pl.ds(NS * RPT, REM)])

        plsc.subcore_barrier()

        def gstart(g, b):
            pltpu.async_copy(
                t_hbm.at[eiA.at[pl.ds(g * C, C)]], rows[b], gsem[b])

        def gwait(g, b):
            pltpu.make_async_copy(
                t_hbm.at[eiA.at[pl.ds(g * C, C)]], rows[b], gsem[b]).wait()

        def sstart(b):
            pltpu.async_copy(rows[b], acc.at[dst[b]], ssem[b], add=True)

        def swait(b):
            pltpu.make_async_copy(rows[b], acc.at[dst[b]], ssem[b]).wait()

        for (src_hbm, dst_hbm) in ((ei0_hbm, ei1_hbm), (ei1_hbm, ei0_hbm)):
            pltpu.sync_copy(src_hbm.at[pl.ds(wid * EW, EW)], eiA)

            def dstart(g, b, dst_hbm=dst_hbm):
                pltpu.async_copy(
                    dst_hbm.at[pl.ds(wid * EW + g * C, C)], dst[b], dsem[b])

            def dwait(g, b, dst_hbm=dst_hbm):
                pltpu.make_async_copy(
                    dst_hbm.at[pl.ds(wid * EW + g * C, C)], dst[b],
                    dsem[b]).wait()

            def process(g, b, wait_prev=True):
                dwait(g, b)
                gwait(g, b)
                if wait_prev:
                    swait((b + RB - 1) % RB)
                sstart(b)

            def launch(g, b):
                dstart(g, b)
                gstart(g, b)

            for k in range(RB - 1):
                launch(k, k)
            for k in range(RB - 1):
                process(k, k, wait_prev=(k > 0))
                launch(k + RB - 1, (k + RB - 1) % RB)

            def body(i, carry):
                g = (RB - 1) + RB * i
                for b in range(RB):
                    k = g + b
                    process(k, (RB - 1 + b) % RB)
                    launch(k + RB - 1, (2 * (RB - 1) + b) % RB)
                return carry

            lax.fori_loop(0, (NCHUNK - 2 * (RB - 1)) // RB, body, 0)
            for k in range(NCHUNK - (RB - 1), NCHUNK):
                process(k, k % RB)
            swait((NCHUNK - 1) % RB)

        plsc.subcore_barrier()
        pltpu.sync_copy(acc.at[pl.ds(s * RPT, RPT)],
                        out_hbm.at[pl.ds(c * N + s * RPT, RPT)])

        @pl.when(s == NS - 1)
        def _out_rem():
            pltpu.sync_copy(acc.at[pl.ds(NS * RPT, REM)],
                            out_hbm.at[pl.ds(c * N + NS * RPT, REM)])

    return agg(t, ei0, ei1)


def _tc_encode(x, We, be, W0, b0):
    D = x.shape[1]

    def body(x_ref, we_ref, be_ref, w0_ref, b0_ref, x0_ref, t_ref):
        x0 = jnp.dot(x_ref[...], we_ref[...],
                     preferred_element_type=jnp.float32) + be_ref[...]
        x0_ref[...] = x0
        t_ref[...] = jnp.dot(x0, w0_ref[...],
                             preferred_element_type=jnp.float32) + b0_ref[...]

    return pl.pallas_call(
        body,
        grid=(G,),
        in_specs=[
            pl.BlockSpec((R, D), lambda i: (i, 0)),
            pl.BlockSpec((D, H), lambda i: (0, 0)),
            pl.BlockSpec((1, H), lambda i: (0, 0)),
            pl.BlockSpec((H, H), lambda i: (0, 0)),
            pl.BlockSpec((1, H), lambda i: (0, 0)),
        ],
        out_specs=(
            pl.BlockSpec((R, H), lambda i: (i, 0)),
            pl.BlockSpec((R, H), lambda i: (i, 0)),
        ),
        out_shape=(
            jax.ShapeDtypeStruct((N, H), jnp.float32),
            jax.ShapeDtypeStruct((N, H), jnp.float32),
        ),
    )(x, We, be.reshape(1, H), W0, b0.reshape(1, H))


def _tc_mid(P, x0, W, b):
    def body(p0_ref, p1_ref, x0_ref, w_ref, b_ref, t_ref):
        h = jnp.maximum(p0_ref[...] + p1_ref[...] + x0_ref[...], 0.0)
        t_ref[...] = jnp.dot(h, w_ref[...],
                             preferred_element_type=jnp.float32) + b_ref[...]

    return pl.pallas_call(
        body,
        grid=(G,),
        in_specs=[
            pl.BlockSpec((R, H), lambda i: (i, 0)),
            pl.BlockSpec((R, H), lambda i: (i + G, 0)),
            pl.BlockSpec((R, H), lambda i: (i, 0)),
            pl.BlockSpec((H, H), lambda i: (0, 0)),
            pl.BlockSpec((1, H), lambda i: (0, 0)),
        ],
        out_specs=pl.BlockSpec((R, H), lambda i: (i, 0)),
        out_shape=jax.ShapeDtypeStruct((N, H), jnp.float32),
    )(P, P, x0, W, b.reshape(1, H))


def _tc_final(P, x0, Wd, bd):
    def body(p0_ref, p1_ref, x0_ref, wd_ref, bd_ref, o_ref):
        h = jnp.maximum(p0_ref[...] + p1_ref[...] + x0_ref[...], 0.0)
        o_ref[...] = jnp.sum(h * wd_ref[...], axis=1,
                             keepdims=True) + bd_ref[...]

    return pl.pallas_call(
        body,
        grid=(G,),
        in_specs=[
            pl.BlockSpec((R, H), lambda i: (i, 0)),
            pl.BlockSpec((R, H), lambda i: (i + G, 0)),
            pl.BlockSpec((R, H), lambda i: (i, 0)),
            pl.BlockSpec((1, H), lambda i: (0, 0)),
            pl.BlockSpec((1, 1), lambda i: (0, 0)),
        ],
        out_specs=pl.BlockSpec((R, 1), lambda i: (i, 0)),
        out_shape=jax.ShapeDtypeStruct((N, 1), jnp.float32),
    )(P, P, x0, Wd.reshape(1, H), bd.reshape(1, 1))


def kernel(x, edge_index, W_enc, b_enc, W_layers, b_layers, W_dec, b_dec):
    L = W_layers.shape[0]
    ei0 = edge_index[0]
    ei1 = edge_index[1]
    x0, t = _tc_encode(x, W_enc, b_enc, W_layers[0], b_layers[0])
    out = None
    for i in range(L):
        P = _sc_aggregate(t, ei0, ei1)
        if i + 1 < L:
            t = _tc_mid(P, x0, W_layers[i + 1], b_layers[i + 1])
        else:
            out = _tc_final(P, x0, W_dec, b_dec)
    return out
